# Initial kernel scaffold; baseline (speedup 1.0000x reference)
import functools

import jax
import jax.numpy as jnp
from jax import lax
from jax.experimental import pallas as pl
from jax.experimental.pallas import tpu as pltpu
from jax.experimental.pallas import tpu_sc as plsc

NN = 10000
D = 128
E = 320000
NC = 2
NS = 16
NW = NC * NS
EPT = E // NW
CH = 16
NCHUNK = EPT // CH
ROWS_PER_TILE = NN // NS
ZROWS = 125

_MESH = plsc.VectorSubcoreMesh(core_axis_name="c", subcore_axis_name="s")


def _zero_1d(ref, n):
    z = jnp.zeros((16,), ref.dtype)

    def body(i, carry):
        ref[pl.ds(i * 16, 16)] = z
        return carry

    lax.fori_loop(0, n // 16, body, 0)


def _sc_degrees_body(src_hbm, dst_hbm, out_hbm, srcv, dstv, hs, hd):
    cid = lax.axis_index("c")
    sid = lax.axis_index("s")
    wid = cid * NS + sid
    base = wid * EPT

    pltpu.sync_copy(src_hbm.at[pl.ds(base, EPT)], srcv)
    pltpu.sync_copy(dst_hbm.at[pl.ds(base, EPT)], dstv)
    _zero_1d(hs, NN)
    _zero_1d(hd, NN)

    ones = jnp.ones((CH,), jnp.float32)

    def body(i, carry):
        s16 = srcv[pl.ds(i * CH, CH)]
        d16 = dstv[pl.ds(i * CH, CH)]
        plsc.addupdate_scatter(hs, [s16], ones)
        plsc.addupdate_scatter(hd, [d16], ones)
        return carry

    lax.fori_loop(0, NCHUNK, body, 0)

    pltpu.sync_copy(hs, out_hbm.at[wid, 0])
    pltpu.sync_copy(hd, out_hbm.at[wid, 1])


_sc_degrees = pl.kernel(
    _sc_degrees_body,
    out_type=jax.ShapeDtypeStruct((NW, 2, NN), jnp.float32),
    mesh=_MESH,
    scratch_types=[
        pltpu.VMEM((EPT,), jnp.int32),
        pltpu.VMEM((EPT,), jnp.int32),
        pltpu.VMEM((NN,), jnp.float32),
        pltpu.VMEM((NN,), jnp.float32),
    ],
)


def _sc_sweep_body(x_hbm, src_hbm, dst_hbm, ew_hbm, do_hbm, di_hbm, out_hbm,
                   srcv, dstv, ewv, dov, div, av, rows, zbuf, acc):
    cid = lax.axis_index("c")
    sid = lax.axis_index("s")
    wid = cid * NS + sid
    base = wid * EPT

    pltpu.sync_copy(src_hbm.at[pl.ds(base, EPT)], srcv)
    pltpu.sync_copy(dst_hbm.at[pl.ds(base, EPT)], dstv)
    pltpu.sync_copy(ew_hbm.at[pl.ds(base, EPT)], ewv)
    pltpu.sync_copy(do_hbm, dov)
    pltpu.sync_copy(di_hbm, div)

    z = jnp.zeros((16,), jnp.float32)

    def zb(i, carry):
        for c in range(8):
            zbuf[i, pl.ds(c * 16, 16)] = z
        return carry

    lax.fori_loop(0, ZROWS, zb, 0)
    for r in range(ROWS_PER_TILE // ZROWS):
        pltpu.sync_copy(
            zbuf, acc.at[pl.ds(sid * ROWS_PER_TILE + r * ZROWS, ZROWS)])
    plsc.subcore_barrier()

    def body(i, carry):
        s16 = srcv[pl.ds(i * CH, CH)]
        d16 = dstv[pl.ds(i * CH, CH)]
        w16 = ewv[pl.ds(i * CH, CH)]
        dos = plsc.load_gather(dov, [s16])
        dis = plsc.load_gather(div, [d16])
        av[...] = w16 * dos * dis
        pltpu.sync_copy(x_hbm.at[s16], rows)
        for r in range(CH):
            sc = av[r]
            for c in range(8):
                rows[r, pl.ds(c * 16, 16)] = rows[r, pl.ds(c * 16, 16)] * sc
        pltpu.sync_copy(rows, acc.at[d16], add=True)
        return carry

    lax.fori_loop(0, NCHUNK, body, 0)
    plsc.subcore_barrier()

    pltpu.sync_copy(acc.at[pl.ds(sid * ROWS_PER_TILE, ROWS_PER_TILE)],
                    out_hbm.at[cid, pl.ds(sid * ROWS_PER_TILE, ROWS_PER_TILE)])


_sc_sweep = pl.kernel(
    _sc_sweep_body,
    out_type=jax.ShapeDtypeStruct((NC, NN, D), jnp.float32),
    mesh=_MESH,
    scratch_types=[
        pltpu.VMEM((EPT,), jnp.int32),
        pltpu.VMEM((EPT,), jnp.int32),
        pltpu.VMEM((EPT,), jnp.float32),
        pltpu.VMEM((NN,), jnp.float32),
        pltpu.VMEM((NN,), jnp.float32),
        pltpu.VMEM((CH,), jnp.float32),
        pltpu.VMEM((CH, D), jnp.float32),
        pltpu.VMEM((ZROWS, D), jnp.float32),
        pltpu.VMEM_SHARED((NN, D), jnp.float32),
    ],
)


def _tc_degrees_body(hist_ref, d_ref):
    deg = jnp.maximum(jnp.sum(hist_ref[...], axis=0), 1.0)
    d_ref[...] = lax.rsqrt(deg)


def _tc_degrees(hist):
    return pl.pallas_call(
        _tc_degrees_body,
        out_shape=jax.ShapeDtypeStruct((2, NN), jnp.float32),
    )(hist)


def _tc_layer_body(p_ref, w_ref, b_ref, f_ref):
    u = p_ref[0] + p_ref[1]
    f_ref[...] = (
        jnp.dot(u, w_ref[...], preferred_element_type=jnp.float32)
        + b_ref[...][None, :])


def _tc_layer(p, W, b):
    return pl.pallas_call(
        _tc_layer_body,
        out_shape=jax.ShapeDtypeStruct((NN, D), jnp.float32),
    )(p, W, b)


def _tc_epilogue_body(p_ref, x_ref, f1_ref, w_ref, b_ref, cw_ref, cb_ref,
                      out_ref):
    f2 = (
        jnp.dot(p_ref[0] + p_ref[1], w_ref[...],
                preferred_element_type=jnp.float32)
        + b_ref[...][None, :])
    xa = x_ref[...]
    f1 = f1_ref[...]
    Nn = NN // 4
    blocks = ([xa[i * Nn:(i + 1) * Nn] for i in range(4)]
              + [f1[i * Nn:(i + 1) * Nn] for i in range(4)]
              + [f2[i * Nn:(i + 1) * Nn] for i in range(4)])
    for bb in range(2):
        for o in range(2):
            acc = jnp.full((Nn, D), cb_ref[o], jnp.float32)
            for c in range(6):
                acc = acc + cw_ref[o, c] * blocks[bb * 6 + c]
            out_ref[bb, o] = acc


def _tc_epilogue(p2, x, f1, W, b, cw, cb):
    return pl.pallas_call(
        _tc_epilogue_body,
        out_shape=jax.ShapeDtypeStruct((2, 2, NN // 4, D), jnp.float32),
        in_specs=[
            pl.BlockSpec(memory_space=pltpu.VMEM),
            pl.BlockSpec(memory_space=pltpu.VMEM),
            pl.BlockSpec(memory_space=pltpu.VMEM),
            pl.BlockSpec(memory_space=pltpu.VMEM),
            pl.BlockSpec(memory_space=pltpu.VMEM),
            pl.BlockSpec(memory_space=pltpu.SMEM),
            pl.BlockSpec(memory_space=pltpu.SMEM),
        ],
    )(p2, x, f1, W, b, cw, cb)


@jax.jit
def kernel(features, edge_index, edge_weight, W, b, conv_W, conv_b):
    x = features.reshape(NN, D)
    src = edge_index[0]
    dst = edge_index[1]

    hist = _sc_degrees(src, dst)
    d = _tc_degrees(hist)
    do = d[0]
    di = d[1]

    p1 = _sc_sweep(x, src, dst, edge_weight, do, di)
    f1 = _tc_layer(p1, W, b)
    p2 = _sc_sweep(f1, src, dst, edge_weight, do, di)
    out = _tc_epilogue(p2, x, f1, W, b, conv_W[:, :, 0, 0], conv_b)
    return out


# speedup vs baseline: 3.1893x; 3.1893x over previous
import jax
import jax.numpy as jnp
from jax import lax
from jax.experimental import pallas as pl
from jax.experimental.pallas import tpu as pltpu
from jax.experimental.pallas import tpu_sc as plsc

NN = 10000
D = 128
E = 320000
NC = 2
NS = 16
NW = NC * NS
EPT = E // NW
CH = 16
NCHUNK = EPT // CH
NNP = 10240
ROWS_PER_TILE = NNP // NS
ZROWS = 32
SCB = 2000
NSUP = EPT // SCB
CPS = SCB // CH

_MESH = plsc.VectorSubcoreMesh(core_axis_name="c", subcore_axis_name="s")


def _zero_1d(ref, n):
    z = jnp.zeros((16,), ref.dtype)

    def body(i, carry):
        ref[pl.ds(i * 16, 16)] = z
        return carry

    lax.fori_loop(0, n // 16, body, 0)


def _sc_degrees_body(src_hbm, dst_hbm, hs_out, hd_out, srcv, dstv, hs, hd):
    cid = lax.axis_index("c")
    sid = lax.axis_index("s")
    wid = cid * NS + sid
    base = wid * EPT

    pltpu.sync_copy(src_hbm.at[pl.ds(base, EPT)], srcv)
    pltpu.sync_copy(dst_hbm.at[pl.ds(base, EPT)], dstv)
    _zero_1d(hs, NN)
    _zero_1d(hd, NN)

    ones = jnp.ones((CH,), jnp.float32)

    def body(i, carry):
        s16 = srcv[pl.ds(i * CH, CH)]
        d16 = dstv[pl.ds(i * CH, CH)]
        plsc.addupdate_scatter(hs, [s16], ones)
        plsc.addupdate_scatter(hd, [d16], ones)
        return carry

    lax.fori_loop(0, NCHUNK, body, 0)

    pltpu.sync_copy(hs, hs_out.at[pl.ds(wid * NN, NN)])
    pltpu.sync_copy(hd, hd_out.at[pl.ds(wid * NN, NN)])


_sc_degrees = pl.kernel(
    _sc_degrees_body,
    out_type=(jax.ShapeDtypeStruct((NW * NN,), jnp.float32),
              jax.ShapeDtypeStruct((NW * NN,), jnp.float32)),
    mesh=_MESH,
    compiler_params=pltpu.CompilerParams(needs_layout_passes=False),
    scratch_types=[
        pltpu.VMEM((EPT,), jnp.int32),
        pltpu.VMEM((EPT,), jnp.int32),
        pltpu.VMEM((NN,), jnp.float32),
        pltpu.VMEM((NN,), jnp.float32),
    ],
)


def _sc_sweep_body(x_hbm, src_hbm, dst2_hbm, ew_hbm, out_hbm,
                   srcv, ewv, dstm, rows, zbuf, acc):
    cid = lax.axis_index("c")
    sid = lax.axis_index("s")
    wid = cid * NS + sid
    base = wid * EPT

    z = jnp.zeros((16,), jnp.float32)

    def zb(i, carry):
        for c in range(8):
            zbuf[i, pl.ds(c * 16, 16)] = z
        return carry

    lax.fori_loop(0, ZROWS, zb, 0)

    def zcp(r, carry):
        pltpu.sync_copy(
            zbuf, acc.at[pl.ds(sid * ROWS_PER_TILE + r * ZROWS, ZROWS)])
        return carry

    lax.fori_loop(0, ROWS_PER_TILE // ZROWS, zcp, 0)
    plsc.subcore_barrier()

    def sup(s, carry):
        off = base + s * SCB
        pltpu.sync_copy(src_hbm.at[pl.ds(off, SCB)], srcv)
        pltpu.sync_copy(ew_hbm.at[pl.ds(off, SCB)], ewv)
        pltpu.sync_copy(dst2_hbm.at[wid * NSUP + s], dstm)

        def body(i, c2):
            s16 = srcv[pl.ds(i * CH, CH)]
            w16 = ewv[pl.ds(i * CH, CH)]
            pltpu.sync_copy(x_hbm.at[s16], rows)
            for r in range(CH):
                sc = w16[r]
                for c in range(8):
                    rows[r, pl.ds(c * 16, 16)] = (
                        rows[r, pl.ds(c * 16, 16)] * sc)
            pltpu.sync_copy(rows, acc.at[dstm.at[i]], add=True)
            return c2

        lax.fori_loop(0, CPS, body, 0)
        return carry

    lax.fori_loop(0, NSUP, sup, 0)
    plsc.subcore_barrier()

    pltpu.sync_copy(acc.at[pl.ds(sid * ROWS_PER_TILE, ROWS_PER_TILE)],
                    out_hbm.at[cid, pl.ds(sid * ROWS_PER_TILE, ROWS_PER_TILE)])


_sc_sweep = pl.kernel(
    _sc_sweep_body,
    out_type=jax.ShapeDtypeStruct((NC, NNP, D), jnp.float32),
    mesh=_MESH,
    compiler_params=pltpu.CompilerParams(needs_layout_passes=False),
    scratch_types=[
        pltpu.VMEM((SCB,), jnp.int32),
        pltpu.VMEM((SCB,), jnp.float32),
        pltpu.VMEM((CPS, CH), jnp.int32),
        pltpu.VMEM((CH, D), jnp.float32),
        pltpu.VMEM((ZROWS, D), jnp.float32),
        pltpu.VMEM_SHARED((NNP, D), jnp.float32),
    ],
)


def _tc_degrees_body(hs_ref, hd_ref, x_ref, do_ref, di_ref, xs_ref):
    do = lax.rsqrt(jnp.maximum(jnp.sum(hs_ref[...], axis=0), 1.0))
    di = lax.rsqrt(jnp.maximum(jnp.sum(hd_ref[...], axis=0), 1.0))
    do_ref[...] = do
    di_ref[...] = di
    xs_ref[...] = x_ref[...] * do[:, None]


def _tc_degrees(hs, hd, x):
    return pl.pallas_call(
        _tc_degrees_body,
        out_shape=(jax.ShapeDtypeStruct((NN,), jnp.float32),
                   jax.ShapeDtypeStruct((NN,), jnp.float32),
                   jax.ShapeDtypeStruct((NN, D), jnp.float32)),
    )(hs, hd, x)


def _tc_layer_body(p_ref, w_ref, b_ref, do_ref, di_ref, f_ref, fs_ref):
    u = (p_ref[0, :NN] + p_ref[1, :NN]) * di_ref[...][:, None]
    f = (jnp.dot(u, w_ref[...], preferred_element_type=jnp.float32)
         + b_ref[...][None, :])
    f_ref[...] = f
    fs_ref[...] = f * do_ref[...][:, None]


def _tc_layer(p, W, b, do, di):
    return pl.pallas_call(
        _tc_layer_body,
        out_shape=(jax.ShapeDtypeStruct((NN, D), jnp.float32),
                   jax.ShapeDtypeStruct((NN, D), jnp.float32)),
    )(p, W, b, do, di)


def _tc_epilogue_body(p_ref, x_ref, f1_ref, w_ref, b_ref, di_ref, cw_ref,
                      cb_ref, out_ref):
    u = (p_ref[0, :NN] + p_ref[1, :NN]) * di_ref[...][:, None]
    f2 = (jnp.dot(u, w_ref[...], preferred_element_type=jnp.float32)
          + b_ref[...][None, :])
    xa = x_ref[...]
    f1 = f1_ref[...]
    Nn = NN // 4
    blocks = ([xa[i * Nn:(i + 1) * Nn] for i in range(4)]
              + [f1[i * Nn:(i + 1) * Nn] for i in range(4)]
              + [f2[i * Nn:(i + 1) * Nn] for i in range(4)])
    for bb in range(2):
        for o in range(2):
            acc = jnp.full((Nn, D), cb_ref[o], jnp.float32)
            for c in range(6):
                acc = acc + cw_ref[o, c] * blocks[bb * 6 + c]
            out_ref[bb, o] = acc


def _tc_epilogue(p2, x, f1, W, b, di, cw, cb):
    return pl.pallas_call(
        _tc_epilogue_body,
        out_shape=jax.ShapeDtypeStruct((2, 2, NN // 4, D), jnp.float32),
        in_specs=[
            pl.BlockSpec(memory_space=pltpu.VMEM),
            pl.BlockSpec(memory_space=pltpu.VMEM),
            pl.BlockSpec(memory_space=pltpu.VMEM),
            pl.BlockSpec(memory_space=pltpu.VMEM),
            pl.BlockSpec(memory_space=pltpu.VMEM),
            pl.BlockSpec(memory_space=pltpu.VMEM),
            pl.BlockSpec(memory_space=pltpu.SMEM),
            pl.BlockSpec(memory_space=pltpu.SMEM),
        ],
    )(p2, x, f1, W, b, di, cw, cb)


@jax.jit
def kernel(features, edge_index, edge_weight, W, b, conv_W, conv_b):
    x = features.reshape(NN, D)
    src = edge_index[0]
    dst = edge_index[1]
    dst2 = dst.reshape(NW * NSUP, CPS, CH)

    hs, hd = _sc_degrees(src, dst)
    do, di, xs = _tc_degrees(hs.reshape(NW, NN), hd.reshape(NW, NN), x)

    p1 = _sc_sweep(xs, src, dst2, edge_weight)
    f1, f1s = _tc_layer(p1, W, b, do, di)
    p2 = _sc_sweep(f1s, src, dst2, edge_weight)
    out = _tc_epilogue(p2, x, f1, W, b, di, conv_W[:, :, 0, 0], conv_b)
    return out


# speedup vs baseline: 3.2034x; 1.0044x over previous
import jax
import jax.numpy as jnp
from jax import lax
from jax.experimental import pallas as pl
from jax.experimental.pallas import tpu as pltpu
from jax.experimental.pallas import tpu_sc as plsc

NN = 10000
D = 128
E = 320000
NC = 2
NS = 16
NW = NC * NS
EPT = E // NW
CH = 16
NCHUNK = EPT // CH
NNP = 10240
ROWS_PER_TILE = NNP // NS
ZROWS = 32
SCB = 2000
NSUP = EPT // SCB
CPS = SCB // CH

_MESH = plsc.VectorSubcoreMesh(core_axis_name="c", subcore_axis_name="s")


def _zero_1d(ref, n):
    z = jnp.zeros((16,), ref.dtype)

    def body(i, carry):
        ref[pl.ds(i * 16, 16)] = z
        return carry

    lax.fori_loop(0, n // 16, body, 0)


def _sc_degrees_body(src_hbm, dst_hbm, hs_out, hd_out, srcv, dstv, hs, hd):
    cid = lax.axis_index("c")
    sid = lax.axis_index("s")
    wid = cid * NS + sid
    base = wid * EPT

    pltpu.sync_copy(src_hbm.at[pl.ds(base, EPT)], srcv)
    pltpu.sync_copy(dst_hbm.at[pl.ds(base, EPT)], dstv)
    _zero_1d(hs, NN)
    _zero_1d(hd, NN)

    ones = jnp.ones((CH,), jnp.float32)

    def body(i, carry):
        s16 = srcv[pl.ds(i * CH, CH)]
        d16 = dstv[pl.ds(i * CH, CH)]
        plsc.addupdate_scatter(hs, [s16], ones)
        plsc.addupdate_scatter(hd, [d16], ones)
        return carry

    lax.fori_loop(0, NCHUNK, body, 0)

    pltpu.sync_copy(hs, hs_out.at[pl.ds(wid * NN, NN)])
    pltpu.sync_copy(hd, hd_out.at[pl.ds(wid * NN, NN)])


_sc_degrees = pl.kernel(
    _sc_degrees_body,
    out_type=(jax.ShapeDtypeStruct((NW * NN,), jnp.float32),
              jax.ShapeDtypeStruct((NW * NN,), jnp.float32)),
    mesh=_MESH,
    compiler_params=pltpu.CompilerParams(needs_layout_passes=False),
    scratch_types=[
        pltpu.VMEM((EPT,), jnp.int32),
        pltpu.VMEM((EPT,), jnp.int32),
        pltpu.VMEM((NN,), jnp.float32),
        pltpu.VMEM((NN,), jnp.float32),
    ],
)


def _sc_sweep_body(x_hbm, src_hbm, dst_hbm, ew_hbm, out_hbm,
                   srcv, ewv, dstv, dst16, rows, zbuf, acc):
    cid = lax.axis_index("c")
    sid = lax.axis_index("s")
    wid = cid * NS + sid
    base = wid * EPT

    z = jnp.zeros((16,), jnp.float32)

    def zb(i, carry):
        for c in range(8):
            zbuf[i, pl.ds(c * 16, 16)] = z
        return carry

    lax.fori_loop(0, ZROWS, zb, 0)

    def zcp(r, carry):
        pltpu.sync_copy(
            zbuf, acc.at[pl.ds(sid * ROWS_PER_TILE + r * ZROWS, ZROWS)])
        return carry

    lax.fori_loop(0, ROWS_PER_TILE // ZROWS, zcp, 0)
    plsc.subcore_barrier()

    def sup(s, carry):
        off = base + s * SCB
        pltpu.sync_copy(src_hbm.at[pl.ds(off, SCB)], srcv)
        pltpu.sync_copy(ew_hbm.at[pl.ds(off, SCB)], ewv)
        pltpu.sync_copy(dst_hbm.at[pl.ds(off, SCB)], dstv)

        def body(i, c2):
            s16 = srcv[pl.ds(i * CH, CH)]
            w16 = ewv[pl.ds(i * CH, CH)]
            dst16[...] = dstv[pl.ds(i * CH, CH)]
            pltpu.sync_copy(x_hbm.at[s16], rows)
            for r in range(CH):
                sc = w16[r]
                for c in range(8):
                    rows[r, pl.ds(c * 16, 16)] = (
                        rows[r, pl.ds(c * 16, 16)] * sc)
            pltpu.sync_copy(rows, acc.at[dst16], add=True)
            return c2

        lax.fori_loop(0, CPS, body, 0)
        return carry

    lax.fori_loop(0, NSUP, sup, 0)
    plsc.subcore_barrier()

    pltpu.sync_copy(acc.at[pl.ds(sid * ROWS_PER_TILE, ROWS_PER_TILE)],
                    out_hbm.at[cid, pl.ds(sid * ROWS_PER_TILE, ROWS_PER_TILE)])


_sc_sweep = pl.kernel(
    _sc_sweep_body,
    out_type=jax.ShapeDtypeStruct((NC, NNP, D), jnp.float32),
    mesh=_MESH,
    compiler_params=pltpu.CompilerParams(needs_layout_passes=False),
    scratch_types=[
        pltpu.VMEM((SCB,), jnp.int32),
        pltpu.VMEM((SCB,), jnp.float32),
        pltpu.VMEM((SCB,), jnp.int32),
        pltpu.VMEM((CH,), jnp.int32),
        pltpu.VMEM((CH, D), jnp.float32),
        pltpu.VMEM((ZROWS, D), jnp.float32),
        pltpu.VMEM_SHARED((NNP, D), jnp.float32),
    ],
)


def _tc_degrees_body(hs_ref, hd_ref, x_ref, do_ref, di_ref, xs_ref):
    do = lax.rsqrt(jnp.maximum(jnp.sum(hs_ref[...], axis=0), 1.0))
    di = lax.rsqrt(jnp.maximum(jnp.sum(hd_ref[...], axis=0), 1.0))
    do_ref[...] = do
    di_ref[...] = di
    xs_ref[...] = x_ref[...] * do[:, None]


def _tc_degrees(hs, hd, x):
    return pl.pallas_call(
        _tc_degrees_body,
        out_shape=(jax.ShapeDtypeStruct((NN,), jnp.float32),
                   jax.ShapeDtypeStruct((NN,), jnp.float32),
                   jax.ShapeDtypeStruct((NN, D), jnp.float32)),
    )(hs, hd, x)


def _tc_layer_body(p_ref, w_ref, b_ref, do_ref, di_ref, f_ref, fs_ref):
    u = (p_ref[0, :NN] + p_ref[1, :NN]) * di_ref[...][:, None]
    f = (jnp.dot(u, w_ref[...], preferred_element_type=jnp.float32)
         + b_ref[...][None, :])
    f_ref[...] = f
    fs_ref[...] = f * do_ref[...][:, None]


def _tc_layer(p, W, b, do, di):
    return pl.pallas_call(
        _tc_layer_body,
        out_shape=(jax.ShapeDtypeStruct((NN, D), jnp.float32),
                   jax.ShapeDtypeStruct((NN, D), jnp.float32)),
    )(p, W, b, do, di)


def _tc_epilogue_body(p_ref, x_ref, f1_ref, w_ref, b_ref, di_ref, cw_ref,
                      cb_ref, out_ref):
    u = (p_ref[0, :NN] + p_ref[1, :NN]) * di_ref[...][:, None]
    f2 = (jnp.dot(u, w_ref[...], preferred_element_type=jnp.float32)
          + b_ref[...][None, :])
    xa = x_ref[...]
    f1 = f1_ref[...]
    Nn = NN // 4
    blocks = ([xa[i * Nn:(i + 1) * Nn] for i in range(4)]
              + [f1[i * Nn:(i + 1) * Nn] for i in range(4)]
              + [f2[i * Nn:(i + 1) * Nn] for i in range(4)])
    for bb in range(2):
        for o in range(2):
            acc = jnp.full((Nn, D), cb_ref[o], jnp.float32)
            for c in range(6):
                acc = acc + cw_ref[o, c] * blocks[bb * 6 + c]
            out_ref[bb, o] = acc


def _tc_epilogue(p2, x, f1, W, b, di, cw, cb):
    return pl.pallas_call(
        _tc_epilogue_body,
        out_shape=jax.ShapeDtypeStruct((2, 2, NN // 4, D), jnp.float32),
        in_specs=[
            pl.BlockSpec(memory_space=pltpu.VMEM),
            pl.BlockSpec(memory_space=pltpu.VMEM),
            pl.BlockSpec(memory_space=pltpu.VMEM),
            pl.BlockSpec(memory_space=pltpu.VMEM),
            pl.BlockSpec(memory_space=pltpu.VMEM),
            pl.BlockSpec(memory_space=pltpu.VMEM),
            pl.BlockSpec(memory_space=pltpu.SMEM),
            pl.BlockSpec(memory_space=pltpu.SMEM),
        ],
    )(p2, x, f1, W, b, di, cw, cb)


@jax.jit
def kernel(features, edge_index, edge_weight, W, b, conv_W, conv_b):
    x = features.reshape(NN, D)
    src = edge_index[0]
    dst = edge_index[1]

    hs, hd = _sc_degrees(src, dst)
    do, di, xs = _tc_degrees(hs.reshape(NW, NN), hd.reshape(NW, NN), x)

    p1 = _sc_sweep(xs, src, dst, edge_weight)
    f1, f1s = _tc_layer(p1, W, b, do, di)
    p2 = _sc_sweep(f1s, src, dst, edge_weight)
    out = _tc_epilogue(p2, x, f1, W, b, di, conv_W[:, :, 0, 0], conv_b)
    return out
